# Initial kernel scaffold; baseline (speedup 1.0000x reference)
import jax
import jax.numpy as jnp
from jax.experimental import pallas as pl


def kernel(feat, edge_index, e_w, W, attn_l, attn_r, attn_ew):
    raise NotImplementedError("write your pallas kernel here")


# speedup vs baseline: 10.6303x; 10.6303x over previous
import functools

import jax
import jax.numpy as jnp
from jax import lax
from jax.experimental import pallas as pl
from jax.experimental.pallas import tpu as pltpu
from jax.experimental.pallas import tpu_sc as plsc

N = 10000
E = 320000
D = 128
EW = 4
NP = 10240
NT = 32
EC = 10240
NB = 80
BK = 128
EP = NT * EC
RPT = NP // 16
NROW = 1024
CH = 1024
CB = CH // BK
NCH = EC // CH


def _tc_front(feat_p, W, al, ar, aew, ew_p):
    def body(feat_ref, w_ref, al_ref, ar_ref, aew_ref, ew_ref,
             ft_ref, el_ref, er_ref, ewe_ref):
        ft = jnp.dot(feat_ref[...], w_ref[...],
                     preferred_element_type=jnp.float32)
        ft_ref[...] = ft
        el_ref[...] = jnp.sum(ft * al_ref[...][None, :], axis=1)
        er_ref[...] = jnp.sum(ft * ar_ref[...][None, :], axis=1)
        ewe_ref[...] = jnp.sum(ew_ref[...] * aew_ref[...][None, :], axis=1)

    nblk = NP // NROW
    eblk = EP // nblk
    return pl.pallas_call(
        body,
        grid=(nblk,),
        in_specs=[
            pl.BlockSpec((NROW, D), lambda i: (i, 0)),
            pl.BlockSpec((D, D), lambda i: (0, 0)),
            pl.BlockSpec((D,), lambda i: (0,)),
            pl.BlockSpec((D,), lambda i: (0,)),
            pl.BlockSpec((EW,), lambda i: (0,)),
            pl.BlockSpec((eblk, EW), lambda i: (i, 0)),
        ],
        out_specs=[
            pl.BlockSpec((NROW, D), lambda i: (i, 0)),
            pl.BlockSpec((NROW,), lambda i: (i,)),
            pl.BlockSpec((NROW,), lambda i: (i,)),
            pl.BlockSpec((eblk,), lambda i: (i,)),
        ],
        out_shape=[
            jax.ShapeDtypeStruct((NP, D), jnp.float32),
            jax.ShapeDtypeStruct((NP,), jnp.float32),
            jax.ShapeDtypeStruct((NP,), jnp.float32),
            jax.ShapeDtypeStruct((EP,), jnp.float32),
        ],
    )(feat_p, W, al, ar, aew, ew_p)


def _sc_edge(ft, el, er, ewe, src3, dst3):
    mesh = plsc.VectorSubcoreMesh(core_axis_name="c", subcore_axis_name="s")

    @functools.partial(
        pl.kernel,
        mesh=mesh,
        compiler_params=pltpu.CompilerParams(needs_layout_passes=False),
        out_type=[
            jax.ShapeDtypeStruct((2, NP, D), jnp.float32),
            jax.ShapeDtypeStruct((2, NP), jnp.float32),
        ],
        scratch_types=[
            pltpu.VMEM((NP,), jnp.float32),
            pltpu.VMEM((NP,), jnp.float32),
            pltpu.VMEM((CB, 1, BK), jnp.int32),
            pltpu.VMEM((CB, 1, BK), jnp.int32),
            pltpu.VMEM((CH,), jnp.float32),
            pltpu.VMEM((CH,), jnp.float32),
            pltpu.VMEM((BK, D), jnp.float32),
            pltpu.VMEM_SHARED((NP, D), jnp.float32),
            pltpu.VMEM_SHARED((NP,), jnp.float32),
            pltpu.SemaphoreType.DMA,
        ],
    )
    def k(ft_h, el_h, er_h, ewe_h, src_h, dst_h, ms_h, den_h,
          el_v, er_v, src_c, dst_c, ewe_c, ex_c, rows_v, ms_sh, den_sh, sem):
        c = lax.axis_index("c")
        s = lax.axis_index("s")
        t = c * 16 + s
        ebase = t * EC
        bbase = t * NB

        pltpu.sync_copy(el_h, el_v)
        pltpu.sync_copy(er_h, er_v)

        zero16 = jnp.zeros((16,), jnp.float32)

        def zrows(j, _):
            for kk in range(8):
                rows_v[j, pl.ds(kk * 16, 16)] = zero16
            return 0
        lax.fori_loop(0, BK, zrows, 0)

        def zex(j, _):
            ex_c[pl.ds(j * 16, 16)] = zero16
            return 0
        lax.fori_loop(0, RPT // 16, zex, 0)

        for q in range(RPT // BK):
            pltpu.sync_copy(rows_v, ms_sh.at[pl.ds(s * RPT + q * BK, BK)])
        pltpu.sync_copy(ex_c.at[pl.ds(0, RPT)], den_sh.at[pl.ds(s * RPT, RPT)])
        plsc.subcore_barrier()

        def chunk(ch, _):
            pltpu.sync_copy(src_h.at[pl.ds(bbase + ch * CB, CB)], src_c)
            pltpu.sync_copy(dst_h.at[pl.ds(bbase + ch * CB, CB)], dst_c)
            pltpu.sync_copy(ewe_h.at[pl.ds(ebase + ch * CH, CH)], ewe_c)

            def p1(b, _):
                for j in range(8):
                    off = j * 16
                    s16 = src_c[b, 0, pl.ds(off, 16)]
                    d16 = dst_c[b, 0, pl.ds(off, 16)]
                    v = (plsc.load_gather(el_v, [s16])
                         + plsc.load_gather(er_v, [d16])
                         + ewe_c[pl.ds(b * BK + off, 16)])
                    v = jnp.where(v >= 0.0, v, 0.2 * v)
                    ex = jnp.exp(v)
                    eid = (ebase + ch * CH + b * BK + off
                           + lax.iota(jnp.int32, 16))
                    ex = jnp.where(eid < E, ex, 0.0)
                    ex_c[pl.ds(b * BK + off, 16)] = ex
                return 0
            lax.fori_loop(0, CB, p1, 0)

            def p2(b, _):
                pltpu.async_copy(ft_h.at[src_c.at[b, 0]], rows_v, sem).wait()

                def scale(j, _):
                    ex16 = ex_c[pl.ds(b * BK + j * 16, 16)]
                    for l in range(16):
                        sc = ex16[l]
                        row = j * 16 + l
                        for kk in range(8):
                            sl = pl.ds(kk * 16, 16)
                            rows_v[row, sl] = rows_v[row, sl] * sc
                    return 0
                lax.fori_loop(0, BK // 16, scale, 0)

                pltpu.sync_copy(rows_v, ms_sh.at[dst_c.at[b, 0]], add=True)
                pltpu.sync_copy(ex_c.at[pl.ds(b * BK, BK)],
                                den_sh.at[dst_c.at[b, 0]], add=True)
                return 0
            lax.fori_loop(0, CB, p2, 0)
            return 0
        lax.fori_loop(0, NCH, chunk, 0)
        plsc.subcore_barrier()

        pltpu.sync_copy(ms_sh.at[pl.ds(s * RPT, RPT)],
                        ms_h.at[c, pl.ds(s * RPT, RPT)])
        pltpu.sync_copy(den_sh.at[pl.ds(s * RPT, RPT)],
                        den_h.at[c, pl.ds(s * RPT, RPT)])

    return k(ft, el, er, ewe, src3, dst3)


def _tc_back(ms0, ms1, d0, d1, feat_p):
    def body(m0_ref, m1_ref, d0_ref, d1_ref, f_ref, o_ref):
        m = m0_ref[...] + m1_ref[...]
        dn = d0_ref[...] + d1_ref[...] + 1e-16
        x = m / dn + f_ref[...]
        o_ref[...] = jnp.where(x > 0.0, x, jnp.exp(x) - 1.0)

    nblk = NP // NROW
    return pl.pallas_call(
        body,
        grid=(nblk,),
        in_specs=[
            pl.BlockSpec((NROW, D), lambda i: (i, 0)),
            pl.BlockSpec((NROW, D), lambda i: (i, 0)),
            pl.BlockSpec((NROW, 1), lambda i: (i, 0)),
            pl.BlockSpec((NROW, 1), lambda i: (i, 0)),
            pl.BlockSpec((NROW, D), lambda i: (i, 0)),
        ],
        out_specs=pl.BlockSpec((NROW, D), lambda i: (i, 0)),
        out_shape=jax.ShapeDtypeStruct((NP, D), jnp.float32),
    )(ms0, ms1, d0, d1, feat_p)


def kernel(feat, edge_index, e_w, W, attn_l, attn_r, attn_ew):
    feat_p = jnp.pad(feat, ((0, NP - N), (0, 0)))
    src3 = jnp.pad(edge_index[0], (0, EP - E)).reshape(NT * NB, 1, BK)
    dst3 = jnp.pad(edge_index[1], (0, EP - E)).reshape(NT * NB, 1, BK)
    ew_p = jnp.pad(e_w, ((0, EP - E), (0, 0)))
    al = attn_l.reshape(D)
    ar = attn_r.reshape(D)
    aew = attn_ew.reshape(EW)

    ft, el, er, ewe = _tc_front(feat_p, W, al, ar, aew, ew_p)
    ms, den = _sc_edge(ft, el, er, ewe, src3, dst3)
    out = _tc_back(ms[0], ms[1],
                   den[0].reshape(NP, 1), den[1].reshape(NP, 1), feat_p)
    return out[:N]


# speedup vs baseline: 11.7288x; 1.1033x over previous
import functools

import jax
import jax.numpy as jnp
from jax import lax
from jax.experimental import pallas as pl
from jax.experimental.pallas import tpu as pltpu
from jax.experimental.pallas import tpu_sc as plsc

N = 10000
E = 320000
D = 128
EW = 4
NP = 10240
NT = 32
EC = 10240
BK = 64
NB = EC // BK
EP = NT * EC
RPT = NP // 16
NROW = 1024
CH = 1024
CB = CH // BK
NCH = EC // CH


def _tc_front(feat_p, W, al, ar, aew, ew_p):
    def body(feat_ref, w_ref, al_ref, ar_ref, aew_ref, ew_ref,
             ft_ref, el_ref, er_ref, ewe_ref):
        ft = jnp.dot(feat_ref[...], w_ref[...],
                     preferred_element_type=jnp.float32)
        ft_ref[...] = ft
        el_ref[...] = jnp.sum(ft * al_ref[...][None, :], axis=1)
        er_ref[...] = jnp.sum(ft * ar_ref[...][None, :], axis=1)
        ewe_ref[...] = jnp.sum(ew_ref[...] * aew_ref[...][None, :], axis=1)

    nblk = NP // NROW
    eblk = EP // nblk
    return pl.pallas_call(
        body,
        grid=(nblk,),
        in_specs=[
            pl.BlockSpec((NROW, D), lambda i: (i, 0)),
            pl.BlockSpec((D, D), lambda i: (0, 0)),
            pl.BlockSpec((D,), lambda i: (0,)),
            pl.BlockSpec((D,), lambda i: (0,)),
            pl.BlockSpec((EW,), lambda i: (0,)),
            pl.BlockSpec((eblk, EW), lambda i: (i, 0)),
        ],
        out_specs=[
            pl.BlockSpec((NROW, D), lambda i: (i, 0)),
            pl.BlockSpec((NROW,), lambda i: (i,)),
            pl.BlockSpec((NROW,), lambda i: (i,)),
            pl.BlockSpec((eblk,), lambda i: (i,)),
        ],
        out_shape=[
            jax.ShapeDtypeStruct((NP, D), jnp.float32),
            jax.ShapeDtypeStruct((NP,), jnp.float32),
            jax.ShapeDtypeStruct((NP,), jnp.float32),
            jax.ShapeDtypeStruct((EP,), jnp.float32),
        ],
    )(feat_p, W, al, ar, aew, ew_p)


def _sc_edge(ft, el, er, ewe, src3, dst3):
    mesh = plsc.VectorSubcoreMesh(core_axis_name="c", subcore_axis_name="s")

    @functools.partial(
        pl.kernel,
        mesh=mesh,
        compiler_params=pltpu.CompilerParams(needs_layout_passes=False),
        out_type=[
            jax.ShapeDtypeStruct((2, NP, D), jnp.float32),
            jax.ShapeDtypeStruct((2, NP), jnp.float32),
        ],
        scratch_types=[
            pltpu.VMEM((NP,), jnp.float32),
            pltpu.VMEM((NP,), jnp.float32),
            pltpu.VMEM((CB, 1, BK), jnp.int32),
            pltpu.VMEM((CB, 1, BK), jnp.int32),
            pltpu.VMEM((CH,), jnp.float32),
            pltpu.VMEM((CH,), jnp.float32),
            pltpu.VMEM((BK, D), jnp.float32),
            pltpu.VMEM((BK, D), jnp.float32),
            pltpu.VMEM_SHARED((NP, D), jnp.float32),
            pltpu.VMEM_SHARED((NP,), jnp.float32),
            pltpu.SemaphoreType.DMA,
            pltpu.SemaphoreType.DMA,
            pltpu.SemaphoreType.DMA,
            pltpu.SemaphoreType.DMA,
            pltpu.SemaphoreType.DMA,
        ],
    )
    def k(ft_h, el_h, er_h, ewe_h, src_h, dst_h, ms_h, den_h,
          el_v, er_v, src_c, dst_c, ewe_c, ex_c, rows_a, rows_b,
          ms_sh, den_sh, gsem_a, gsem_b, ssem_a, ssem_b, dsem):
        c = lax.axis_index("c")
        s = lax.axis_index("s")
        t = c * 16 + s
        ebase = t * EC
        bbase = t * NB

        pltpu.sync_copy(el_h, el_v)
        pltpu.sync_copy(er_h, er_v)

        zero16 = jnp.zeros((16,), jnp.float32)

        def zrows(j, _):
            for kk in range(8):
                rows_a[j, pl.ds(kk * 16, 16)] = zero16
            return 0
        lax.fori_loop(0, BK, zrows, 0)

        def zex(j, _):
            ex_c[pl.ds(j * 16, 16)] = zero16
            return 0
        lax.fori_loop(0, RPT // 16, zex, 0)

        for q in range(RPT // BK):
            pltpu.sync_copy(rows_a, ms_sh.at[pl.ds(s * RPT + q * BK, BK)])
        pltpu.sync_copy(ex_c.at[pl.ds(0, RPT)], den_sh.at[pl.ds(s * RPT, RPT)])
        plsc.subcore_barrier()

        def chunk(ch, _):
            pltpu.sync_copy(src_h.at[pl.ds(bbase + ch * CB, CB)], src_c)
            pltpu.sync_copy(dst_h.at[pl.ds(bbase + ch * CB, CB)], dst_c)
            pltpu.sync_copy(ewe_h.at[pl.ds(ebase + ch * CH, CH)], ewe_c)

            def p1(b, _):
                for j in range(BK // 16):
                    off = j * 16
                    s16 = src_c[b, 0, pl.ds(off, 16)]
                    d16 = dst_c[b, 0, pl.ds(off, 16)]
                    v = (plsc.load_gather(el_v, [s16])
                         + plsc.load_gather(er_v, [d16])
                         + ewe_c[pl.ds(b * BK + off, 16)])
                    v = jnp.where(v >= 0.0, v, 0.2 * v)
                    ex = jnp.exp(v)
                    eid = (ebase + ch * CH + b * BK + off
                           + lax.iota(jnp.int32, 16))
                    ex = jnp.where(eid < E, ex, 0.0)
                    ex_c[pl.ds(b * BK + off, 16)] = ex
                return 0
            lax.fori_loop(0, CB, p1, 0)

            rows = (rows_a, rows_b)
            gsem = (gsem_a, gsem_b)
            ssem = (ssem_a, ssem_b)
            pend_g = [None, None]
            pend_s = [None, None]
            pend_d = []
            pend_g[0] = pltpu.async_copy(ft_h.at[src_c.at[0, 0]],
                                         rows[0], gsem[0])
            for b in range(CB):
                pb = b % 2
                nx = (b + 1) % 2
                if b + 1 < CB:
                    if pend_s[nx] is not None:
                        pend_s[nx].wait()
                        pend_s[nx] = None
                    pend_g[nx] = pltpu.async_copy(
                        ft_h.at[src_c.at[b + 1, 0]], rows[nx], gsem[nx])
                pend_g[pb].wait()

                def scale(j, _, b=b, buf=rows[pb]):
                    ex16 = ex_c[pl.ds(b * BK + j * 16, 16)]
                    for l in range(16):
                        sc = ex16[l]
                        row = j * 16 + l
                        for kk in range(8):
                            sl = pl.ds(kk * 16, 16)
                            buf[row, sl] = buf[row, sl] * sc
                    return 0
                lax.fori_loop(0, BK // 16, scale, 0)

                pend_d.append(pltpu.async_copy(
                    ex_c.at[pl.ds(b * BK, BK)],
                    den_sh.at[dst_c.at[b, 0]], dsem, add=True))
                pend_s[pb] = pltpu.async_copy(
                    rows[pb], ms_sh.at[dst_c.at[b, 0]], ssem[pb], add=True)
            for p in pend_s:
                if p is not None:
                    p.wait()
            for p in pend_d:
                p.wait()
            return 0
        lax.fori_loop(0, NCH, chunk, 0)
        plsc.subcore_barrier()

        pltpu.sync_copy(ms_sh.at[pl.ds(s * RPT, RPT)],
                        ms_h.at[c, pl.ds(s * RPT, RPT)])
        pltpu.sync_copy(den_sh.at[pl.ds(s * RPT, RPT)],
                        den_h.at[c, pl.ds(s * RPT, RPT)])

    return k(ft, el, er, ewe, src3, dst3)


def _tc_back(ms0, ms1, d0, d1, feat_p):
    def body(m0_ref, m1_ref, d0_ref, d1_ref, f_ref, o_ref):
        m = m0_ref[...] + m1_ref[...]
        dn = d0_ref[...] + d1_ref[...] + 1e-16
        x = m / dn + f_ref[...]
        o_ref[...] = jnp.where(x > 0.0, x, jnp.exp(x) - 1.0)

    nblk = NP // NROW
    return pl.pallas_call(
        body,
        grid=(nblk,),
        in_specs=[
            pl.BlockSpec((NROW, D), lambda i: (i, 0)),
            pl.BlockSpec((NROW, D), lambda i: (i, 0)),
            pl.BlockSpec((NROW, 1), lambda i: (i, 0)),
            pl.BlockSpec((NROW, 1), lambda i: (i, 0)),
            pl.BlockSpec((NROW, D), lambda i: (i, 0)),
        ],
        out_specs=pl.BlockSpec((NROW, D), lambda i: (i, 0)),
        out_shape=jax.ShapeDtypeStruct((NP, D), jnp.float32),
    )(ms0, ms1, d0, d1, feat_p)


def kernel(feat, edge_index, e_w, W, attn_l, attn_r, attn_ew):
    feat_p = jnp.pad(feat, ((0, NP - N), (0, 0)))
    src3 = jnp.pad(edge_index[0], (0, EP - E)).reshape(NT * NB, 1, BK)
    dst3 = jnp.pad(edge_index[1], (0, EP - E)).reshape(NT * NB, 1, BK)
    ew_p = jnp.pad(e_w, ((0, EP - E), (0, 0)))
    al = attn_l.reshape(D)
    ar = attn_r.reshape(D)
    aew = attn_ew.reshape(EW)

    ft, el, er, ewe = _tc_front(feat_p, W, al, ar, aew, ew_p)
    ms, den = _sc_edge(ft, el, er, ewe, src3, dst3)
    out = _tc_back(ms[0], ms[1],
                   den[0].reshape(NP, 1), den[1].reshape(NP, 1), feat_p)
    return out[:N]
